# baseline (speedup 1.0000x reference)
import functools

import jax
import jax.numpy as jnp
from jax import lax
from jax.experimental import pallas as pl
from jax.experimental.pallas import tpu as pltpu
from jax.experimental.pallas import tpu_sc as plsc

N = 10000
E = 320000
IN_SHAPE = 128
PE_DIM = 8
HID = 128

_BLK = 1000


def _enc_body(x_ref, pe_ref, w_ref, b_ref, w1_ref, b1_ref, w2_ref, b2_ref,
              out_ref):
    x = x_ref[...]
    pe = pe_ref[...]
    h1 = jnp.dot(x, w_ref[...], preferred_element_type=jnp.float32, precision=lax.Precision.HIGHEST) + b_ref[...]
    z = jnp.dot(pe, w1_ref[...], preferred_element_type=jnp.float32, precision=lax.Precision.HIGHEST)
    b1 = b1_ref[...]
    h = jax.nn.relu(z + b1) + jax.nn.relu(-z + b1)
    h2 = 0.5 * jnp.dot(h, w2_ref[...], preferred_element_type=jnp.float32, precision=lax.Precision.HIGHEST) \
        + b2_ref[...]
    out_ref[...] = jax.nn.relu(jnp.concatenate([h1, h2], axis=1))


def _encoder(x, pe, w, b, w1, b1, w2, b2):
    pe_p = jnp.pad(pe, ((0, 0), (0, IN_SHAPE - PE_DIM)))
    w1_p = jnp.pad(w1, ((0, IN_SHAPE - PE_DIM), (0, 0)))
    grid = (N // _BLK,)
    return pl.pallas_call(
        _enc_body,
        grid=grid,
        in_specs=[
            pl.BlockSpec((_BLK, IN_SHAPE), lambda i: (i, 0)),
            pl.BlockSpec((_BLK, IN_SHAPE), lambda i: (i, 0)),
            pl.BlockSpec((IN_SHAPE, HID // 2), lambda i: (0, 0)),
            pl.BlockSpec((1, HID // 2), lambda i: (0, 0)),
            pl.BlockSpec((IN_SHAPE, HID), lambda i: (0, 0)),
            pl.BlockSpec((1, HID), lambda i: (0, 0)),
            pl.BlockSpec((HID, HID // 2), lambda i: (0, 0)),
            pl.BlockSpec((1, HID // 2), lambda i: (0, 0)),
        ],
        out_specs=pl.BlockSpec((_BLK, HID), lambda i: (i, 0)),
        out_shape=jax.ShapeDtypeStruct((N, HID), jnp.float32),
    )(x, pe_p, w, b.reshape(1, -1), w1_p, b1.reshape(1, -1), w2,
      b2.reshape(1, -1))


def _round_body(dn_ref, xd_ref, w1_ref, b1_ref, w2_ref, b2_ref, out_ref):
    dn = dn_ref[...]
    den = jnp.concatenate([dn[0, :, :HID // 2], dn[1, :, :HID // 2]], axis=1)
    num = jnp.concatenate([dn[0, :, HID // 2:], dn[1, :, HID // 2:]], axis=1)
    agg = num / (den + 1e-16)
    o = agg + xd_ref[...]
    h = jax.nn.relu(
        jnp.dot(o, w1_ref[...], preferred_element_type=jnp.float32, precision=lax.Precision.HIGHEST)
        + b1_ref[...])
    out_ref[...] = jnp.dot(h, w2_ref[...],
                           preferred_element_type=jnp.float32, precision=lax.Precision.HIGHEST) + b2_ref[...]


def _round_update(dn, x_dst, w1, b1, w2, b2):
    grid = (N // _BLK,)
    return pl.pallas_call(
        _round_body,
        grid=grid,
        in_specs=[
            pl.BlockSpec((2, _BLK, HID), lambda i: (0, i, 0)),
            pl.BlockSpec((_BLK, HID), lambda i: (i, 0)),
            pl.BlockSpec((HID, 2 * HID), lambda i: (0, 0)),
            pl.BlockSpec((1, 2 * HID), lambda i: (0, 0)),
            pl.BlockSpec((2 * HID, HID), lambda i: (0, 0)),
            pl.BlockSpec((1, HID), lambda i: (0, 0)),
        ],
        out_specs=pl.BlockSpec((_BLK, HID), lambda i: (i, 0)),
        out_shape=jax.ShapeDtypeStruct((N, HID), jnp.float32),
    )(dn, x_dst, w1, b1.reshape(1, -1), w2, b2.reshape(1, -1))


def _pred_body(v_ref, w1_ref, b1_ref, w2_ref, b2_ref, out_ref):
    h = jax.nn.relu(
        jnp.dot(v_ref[...], w1_ref[...], preferred_element_type=jnp.float32, precision=lax.Precision.HIGHEST)
        + b1_ref[...])
    out_ref[...] = jnp.dot(h, w2_ref[...],
                           preferred_element_type=jnp.float32, precision=lax.Precision.HIGHEST) + b2_ref[...]


def _pred(v2, w1, b1, w2, b2):
    w2_p = jnp.pad(w2, ((0, 0), (0, 7)))
    b2_p = jnp.pad(b2, ((0, 7)))
    grid = (2 * N // _BLK,)
    return pl.pallas_call(
        _pred_body,
        grid=grid,
        in_specs=[
            pl.BlockSpec((_BLK, HID), lambda i: (i, 0)),
            pl.BlockSpec((HID, HID), lambda i: (0, 0)),
            pl.BlockSpec((1, HID), lambda i: (0, 0)),
            pl.BlockSpec((HID, 8), lambda i: (0, 0)),
            pl.BlockSpec((1, 8), lambda i: (0, 0)),
        ],
        out_specs=pl.BlockSpec((_BLK, 8), lambda i: (i, 0)),
        out_shape=jax.ShapeDtypeStruct((2 * N, 8), jnp.float32),
    )(v2, w1, b1.reshape(1, -1), w2_p, b2_p.reshape(1, -1))


_HALF = HID // 2
_B = 80
_NCHK = E // _B
_CPS = _NCHK // 16
_RPS = 624
_ZR = 48


def _lane_bcast(v, i):
    return lax.gather(
        v, jnp.full((16, 1), i, jnp.int32),
        lax.GatherDimensionNumbers(offset_dims=(), collapsed_slice_dims=(0,),
                                   start_index_map=(0,)),
        (1,), mode=lax.GatherScatterMode.PROMISE_IN_BOUNDS)


def _sc_agg_body(h2, edata, ewr, wbr, out,
                 eb0, eb1, ewb0, ewb1, ewc, dstv, rows0, rows1, och0, och1,
                 wbv, zv, acc,
                 esem0, esem1, wsem0, wsem1, gsem0, gsem1, ssem0, ssem1):
    c = lax.axis_index("c")
    s = lax.axis_index("s")
    cbase = c * _HALF
    eb = (eb0, eb1)
    ewb = (ewb0, ewb1)
    rows = (rows0, rows1)
    och = (och0, och1)
    esem = (esem0, esem1)
    wsem = (wsem0, wsem1)
    gsem = (gsem0, gsem1)
    ssem = (ssem0, ssem1)

    def zrow(r, carry):
        for j in range(8):
            zv[r, pl.ds(16 * j, 16)] = jnp.zeros((16,), jnp.float32)
        return carry
    lax.fori_loop(0, _ZR, zrow, 0)
    row0 = s * _RPS
    for k in range(_RPS // _ZR):
        pltpu.sync_copy(zv, acc.at[pl.ds(row0 + k * _ZR, _ZR)])

    @pl.when(s == 15)
    def _zero_tail():
        pltpu.sync_copy(zv.at[pl.ds(0, 16)], acc.at[pl.ds(N - 16, 16)])
    plsc.subcore_barrier()

    pltpu.sync_copy(wbr.at[c], wbv)
    wev = [wbv[pl.ds(16 * j, 16)] for j in range(4)]
    bev = [wbv[pl.ds(_HALF + 16 * j, 16)] for j in range(4)]

    crow = s * _CPS

    def e_issue(i, p):
        pltpu.async_copy(edata.at[crow + i], eb[p], esem[p])
        pltpu.async_copy(ewr.at[crow + i], ewb[p], wsem[p])

    def e_wait(i, p):
        pltpu.make_async_copy(edata.at[crow + i], eb[p], esem[p]).wait()
        pltpu.make_async_copy(ewr.at[crow + i], ewb[p], wsem[p]).wait()

    def g_issue(p):
        pltpu.async_copy(h2.at[eb[p].at[0]], rows[p], gsem[p])

    def g_wait(p):
        pltpu.make_async_copy(h2.at[eb[p].at[0]], rows[p], gsem[p]).wait()

    def s_issue(p):
        pltpu.async_copy(och[p], acc.at[dstv.at[p]], ssem[p], add=True)

    def s_wait(p):
        pltpu.make_async_copy(och[p], acc.at[dstv.at[p]], ssem[p]).wait()

    e_issue(0, 0)
    e_issue(1, 1)
    e_wait(0, 0)
    g_issue(0)

    def step(i, p):
        g_wait(p)

        @pl.when(i >= 2)
        def _drain():
            s_wait(p)

        for t in range(_B // 16):
            sl = pl.ds(16 * t, 16)
            dstv[p, sl] = eb[p][1, sl]
            ewc[p, sl] = ewb[p][sl]

        @pl.when(i + 2 < _CPS)
        def _stage():
            e_issue(i + 2, p)

        @pl.when(i + 1 < _CPS)
        def _next_gather():
            e_wait(i + 1, 1 - p)
            g_issue(1 - p)

        @plsc.parallel_loop(0, _B, 1, unroll=8)
        def _edge(e):
            g = e & ~15
            ewv = ewc[p, pl.ds(g, 16)]
            w = _lane_bcast(ewv, e - g)
            for j in range(4):
                a = w * wev[j] + bev[j]
                v = rows[p][e, pl.ds(cbase + 16 * j, 16)]
                m = jnp.maximum(v + a, 1e-7)
                x = jnp.exp(m)
                och[p][e, pl.ds(16 * j, 16)] = x
                och[p][e, pl.ds(_HALF + 16 * j, 16)] = x * m
        s_issue(p)

    def pair(k, carry):
        step(2 * k, 0)
        step(2 * k + 1, 1)
        return carry
    lax.fori_loop(0, _CPS // 2, pair, 0)

    s_wait(0)
    s_wait(1)

    plsc.subcore_barrier()
    for k in range(_RPS // _ZR):
        r = row0 + k * _ZR
        pltpu.sync_copy(acc.at[pl.ds(r, _ZR)], out.at[c, pl.ds(r, _ZR)])

    @pl.when(s == 15)
    def _write_tail():
        pltpu.sync_copy(acc.at[pl.ds(N - 16, 16)],
                        out.at[c, pl.ds(N - 16, 16)])


@functools.cache
def _make_sc_agg():
    mesh = plsc.VectorSubcoreMesh(core_axis_name="c", subcore_axis_name="s",
                                  num_cores=2, num_subcores=16)
    return pl.kernel(
        _sc_agg_body,
        mesh=mesh,
        out_type=jax.ShapeDtypeStruct((2, N, HID), jnp.float32),
        scratch_types=[
            pltpu.VMEM((2, _B), jnp.int32),
            pltpu.VMEM((2, _B), jnp.int32),
            pltpu.VMEM((_B,), jnp.float32),
            pltpu.VMEM((_B,), jnp.float32),
            pltpu.VMEM((2, _B), jnp.float32),
            pltpu.VMEM((2, _B), jnp.int32),
            pltpu.VMEM((_B, HID), jnp.float32),
            pltpu.VMEM((_B, HID), jnp.float32),
            pltpu.VMEM((_B, HID), jnp.float32),
            pltpu.VMEM((_B, HID), jnp.float32),
            pltpu.VMEM((HID,), jnp.float32),
            pltpu.VMEM((_ZR, HID), jnp.float32),
            pltpu.VMEM_SHARED((N, HID), jnp.float32),
            pltpu.SemaphoreType.DMA,
            pltpu.SemaphoreType.DMA,
            pltpu.SemaphoreType.DMA,
            pltpu.SemaphoreType.DMA,
            pltpu.SemaphoreType.DMA,
            pltpu.SemaphoreType.DMA,
            pltpu.SemaphoreType.DMA,
            pltpu.SemaphoreType.DMA,
        ],
    )


def _pack_edges(src, dst, ew):
    edata = jnp.stack([src.reshape(-1, _B), dst.reshape(-1, _B)],
                      axis=1)
    return edata, ew.reshape(-1, _B)


def _aggregate(h_src, edata, ewr, we, be):
    bep = be + 1e-7
    wb = jnp.stack([
        jnp.concatenate([we[0, :_HALF], bep[:_HALF]]),
        jnp.concatenate([we[0, _HALF:], bep[_HALF:]]),
    ])
    return _make_sc_agg()(h_src, edata, ewr, wb)


def kernel(x_vals, x_cons, pe_vals, pe_cons, edge_index_v2c, edge_weight_v2c,
           edge_index_c2v, edge_weight_c2v, enc_vals_W, enc_vals_b,
           pe_vals_W1, pe_vals_b1, pe_vals_W2, pe_vals_b2,
           pred_vals_W1, pred_vals_b1, pred_vals_W2, pred_vals_b2,
           enc_cons_W, enc_cons_b,
           pe_cons_W1, pe_cons_b1, pe_cons_W2, pe_cons_b2,
           pred_cons_W1, pred_cons_b1, pred_cons_W2, pred_cons_b2,
           v2c0_We, v2c0_be, v2c0_W1, v2c0_b1, v2c0_W2, v2c0_b2,
           c2v0_We, c2v0_be, c2v0_W1, c2v0_b1, c2v0_W2, c2v0_b2,
           v2c1_We, v2c1_be, v2c1_W1, v2c1_b1, v2c1_W2, v2c1_b2,
           c2v1_We, c2v1_be, c2v1_W1, c2v1_b1, c2v1_W2, c2v1_b2):
    vals = _encoder(x_vals, pe_vals, enc_vals_W, enc_vals_b,
                    pe_vals_W1, pe_vals_b1, pe_vals_W2, pe_vals_b2)
    cons = _encoder(x_cons, pe_cons, enc_cons_W, enc_cons_b,
                    pe_cons_W1, pe_cons_b1, pe_cons_W2, pe_cons_b2)

    ed_v2c, ewr_v2c = _pack_edges(edge_index_v2c[0], edge_index_v2c[1],
                                  edge_weight_v2c[:, 0])
    ed_c2v, ewr_c2v = _pack_edges(edge_index_c2v[0], edge_index_c2v[1],
                                  edge_weight_c2v[:, 0])

    rounds = [
        ("v2c", v2c0_We, v2c0_be, v2c0_W1, v2c0_b1, v2c0_W2, v2c0_b2),
        ("c2v", c2v0_We, c2v0_be, c2v0_W1, c2v0_b1, c2v0_W2, c2v0_b2),
        ("v2c", v2c1_We, v2c1_be, v2c1_W1, v2c1_b1, v2c1_W2, v2c1_b2),
        ("c2v", c2v1_We, c2v1_be, c2v1_W1, c2v1_b1, c2v1_W2, c2v1_b2),
    ]
    hid_v, hid_c = [], []
    for d, we, be, w1, b1, w2, b2 in rounds:
        if d == "v2c":
            dn = _aggregate(vals, ed_v2c, ewr_v2c, we, be)
            cons = _round_update(dn, cons, w1, b1, w2, b2)
            hid_c.append(cons)
        else:
            dn = _aggregate(cons, ed_c2v, ewr_c2v, we, be)
            vals = _round_update(dn, vals, w1, b1, w2, b2)
            hid_v.append(vals)

    v2 = jnp.concatenate(hid_v, axis=0)
    c2 = jnp.concatenate(hid_c, axis=0)
    pv = _pred(v2, pred_vals_W1, pred_vals_b1, pred_vals_W2, pred_vals_b2)
    pc = _pred(c2, pred_cons_W1, pred_cons_b1, pred_cons_W2, pred_cons_b2)
    v = pv[:, 0].reshape(2, N).T
    c = pc[:, 0].reshape(2, N).T
    return (v, c)


# speedup vs baseline: 1.2054x; 1.2054x over previous
import functools

import jax
import jax.numpy as jnp
from jax import lax
from jax.experimental import pallas as pl
from jax.experimental.pallas import tpu as pltpu
from jax.experimental.pallas import tpu_sc as plsc

N = 10000
E = 320000
IN_SHAPE = 128
PE_DIM = 8
HID = 128

_BLK = 1000


def _enc_body(x_ref, pe_ref, w_ref, b_ref, w1_ref, b1_ref, w2_ref, b2_ref,
              out_ref):
    x = x_ref[...]
    pe = pe_ref[...]
    h1 = jnp.dot(x, w_ref[...], preferred_element_type=jnp.float32, precision=lax.Precision.HIGHEST) + b_ref[...]
    z = jnp.dot(pe, w1_ref[...], preferred_element_type=jnp.float32, precision=lax.Precision.HIGHEST)
    b1 = b1_ref[...]
    h = jax.nn.relu(z + b1) + jax.nn.relu(-z + b1)
    h2 = 0.5 * jnp.dot(h, w2_ref[...], preferred_element_type=jnp.float32, precision=lax.Precision.HIGHEST) \
        + b2_ref[...]
    out_ref[...] = jax.nn.relu(jnp.concatenate([h1, h2], axis=1))


def _encoder(x, pe, w, b, w1, b1, w2, b2):
    pe_p = jnp.pad(pe, ((0, 0), (0, IN_SHAPE - PE_DIM)))
    w1_p = jnp.pad(w1, ((0, IN_SHAPE - PE_DIM), (0, 0)))
    grid = (N // _BLK,)
    return pl.pallas_call(
        _enc_body,
        grid=grid,
        in_specs=[
            pl.BlockSpec((_BLK, IN_SHAPE), lambda i: (i, 0)),
            pl.BlockSpec((_BLK, IN_SHAPE), lambda i: (i, 0)),
            pl.BlockSpec((IN_SHAPE, HID // 2), lambda i: (0, 0)),
            pl.BlockSpec((1, HID // 2), lambda i: (0, 0)),
            pl.BlockSpec((IN_SHAPE, HID), lambda i: (0, 0)),
            pl.BlockSpec((1, HID), lambda i: (0, 0)),
            pl.BlockSpec((HID, HID // 2), lambda i: (0, 0)),
            pl.BlockSpec((1, HID // 2), lambda i: (0, 0)),
        ],
        out_specs=pl.BlockSpec((_BLK, HID), lambda i: (i, 0)),
        out_shape=jax.ShapeDtypeStruct((N, HID), jnp.float32),
    )(x, pe_p, w, b.reshape(1, -1), w1_p, b1.reshape(1, -1), w2,
      b2.reshape(1, -1))


def _round_body(dn_ref, xd_ref, w1_ref, b1_ref, w2_ref, b2_ref, out_ref):
    dn = dn_ref[...]
    den = jnp.concatenate([dn[0, :, :HID // 2], dn[1, :, :HID // 2]], axis=1)
    num = jnp.concatenate([dn[0, :, HID // 2:], dn[1, :, HID // 2:]], axis=1)
    agg = num / (den + 1e-16)
    o = agg + xd_ref[...]
    h = jax.nn.relu(
        jnp.dot(o, w1_ref[...], preferred_element_type=jnp.float32, precision=lax.Precision.HIGHEST)
        + b1_ref[...])
    out_ref[...] = jnp.dot(h, w2_ref[...],
                           preferred_element_type=jnp.float32, precision=lax.Precision.HIGHEST) + b2_ref[...]


def _round_update(dn, x_dst, w1, b1, w2, b2):
    grid = (N // _BLK,)
    return pl.pallas_call(
        _round_body,
        grid=grid,
        in_specs=[
            pl.BlockSpec((2, _BLK, HID), lambda i: (0, i, 0)),
            pl.BlockSpec((_BLK, HID), lambda i: (i, 0)),
            pl.BlockSpec((HID, 2 * HID), lambda i: (0, 0)),
            pl.BlockSpec((1, 2 * HID), lambda i: (0, 0)),
            pl.BlockSpec((2 * HID, HID), lambda i: (0, 0)),
            pl.BlockSpec((1, HID), lambda i: (0, 0)),
        ],
        out_specs=pl.BlockSpec((_BLK, HID), lambda i: (i, 0)),
        out_shape=jax.ShapeDtypeStruct((N, HID), jnp.float32),
    )(dn, x_dst, w1, b1.reshape(1, -1), w2, b2.reshape(1, -1))


def _pred_body(v_ref, w1_ref, b1_ref, w2_ref, b2_ref, out_ref):
    h = jax.nn.relu(
        jnp.dot(v_ref[...], w1_ref[...], preferred_element_type=jnp.float32, precision=lax.Precision.HIGHEST)
        + b1_ref[...])
    out_ref[...] = jnp.dot(h, w2_ref[...],
                           preferred_element_type=jnp.float32, precision=lax.Precision.HIGHEST) + b2_ref[...]


def _pred(v2, w1, b1, w2, b2):
    w2_p = jnp.pad(w2, ((0, 0), (0, 7)))
    b2_p = jnp.pad(b2, ((0, 7)))
    grid = (2 * N // _BLK,)
    return pl.pallas_call(
        _pred_body,
        grid=grid,
        in_specs=[
            pl.BlockSpec((_BLK, HID), lambda i: (i, 0)),
            pl.BlockSpec((HID, HID), lambda i: (0, 0)),
            pl.BlockSpec((1, HID), lambda i: (0, 0)),
            pl.BlockSpec((HID, 8), lambda i: (0, 0)),
            pl.BlockSpec((1, 8), lambda i: (0, 0)),
        ],
        out_specs=pl.BlockSpec((_BLK, 8), lambda i: (i, 0)),
        out_shape=jax.ShapeDtypeStruct((2 * N, 8), jnp.float32),
    )(v2, w1, b1.reshape(1, -1), w2_p, b2_p.reshape(1, -1))


_HALF = HID // 2
_B = 80
_NCHK = E // _B
_CPS = _NCHK // 16
_RPS = 624
_ZR = 48


def _lane_bcast(v, i):
    return lax.gather(
        v, jnp.full((16, 1), i, jnp.int32),
        lax.GatherDimensionNumbers(offset_dims=(), collapsed_slice_dims=(0,),
                                   start_index_map=(0,)),
        (1,), mode=lax.GatherScatterMode.PROMISE_IN_BOUNDS)


def _sc_agg_body(h2, edata, ewr, wbr, out,
                 eb0, eb1, ewb0, ewb1, ewc, dstv, gi, rows0, rows1,
                 och0, och1, wbv, zv, acc,
                 esem0, esem1, wsem0, wsem1, gsem0, gsem1, ssem0, ssem1):
    c = lax.axis_index("c")
    s = lax.axis_index("s")
    cbase = c * _HALF
    eb = (eb0, eb1)
    ewb = (ewb0, ewb1)
    rows = (rows0, rows1)
    och = (och0, och1)
    esem = (esem0, esem1)
    wsem = (wsem0, wsem1)
    gsem = (gsem0, gsem1)
    ssem = (ssem0, ssem1)

    def zrow(r, carry):
        for j in range(8):
            zv[r, pl.ds(16 * j, 16)] = jnp.zeros((16,), jnp.float32)
        return carry
    lax.fori_loop(0, _ZR, zrow, 0)
    row0 = s * _RPS
    for k in range(_RPS // _ZR):
        pltpu.sync_copy(zv, acc.at[pl.ds(row0 + k * _ZR, _ZR)])

    @pl.when(s == 15)
    def _zero_tail():
        pltpu.sync_copy(zv.at[pl.ds(0, 16)], acc.at[pl.ds(N - 16, 16)])
    plsc.subcore_barrier()

    pltpu.sync_copy(wbr.at[c], wbv)
    wev = [wbv[pl.ds(16 * j, 16)] for j in range(4)]
    bev = [wbv[pl.ds(_HALF + 16 * j, 16)] for j in range(4)]

    crow = s * _CPS

    def e_issue(i, p):
        pltpu.async_copy(edata.at[crow + i], eb[p], esem[p])
        pltpu.async_copy(ewr.at[crow + i], ewb[p], wsem[p])

    def e_wait(i, p):
        pltpu.make_async_copy(edata.at[crow + i], eb[p], esem[p]).wait()
        pltpu.make_async_copy(ewr.at[crow + i], ewb[p], wsem[p]).wait()

    def g_issue(p):
        pltpu.async_copy(h2.at[gi.at[p]], rows[p], gsem[p])

    def g_wait(p):
        pltpu.make_async_copy(h2.at[gi.at[p]], rows[p], gsem[p]).wait()

    def s_issue(p):
        pltpu.async_copy(och[p], acc.at[dstv.at[p]], ssem[p], add=True)

    def s_wait(p):
        pltpu.make_async_copy(och[p], acc.at[dstv.at[p]], ssem[p]).wait()

    def fill_gi(p):
        for t in range(_B // 16):
            sl = pl.ds(16 * t, 16)
            gi[p, sl] = eb[p][0, sl] * 2 + c

    e_issue(0, 0)
    e_issue(1, 1)
    e_wait(0, 0)
    fill_gi(0)
    g_issue(0)

    def step(i, p):
        g_wait(p)

        @pl.when(i >= 2)
        def _drain():
            s_wait(p)

        for t in range(_B // 16):
            sl = pl.ds(16 * t, 16)
            dstv[p, sl] = eb[p][1, sl]
            ewc[p, sl] = ewb[p][sl]

        @pl.when(i + 2 < _CPS)
        def _stage():
            e_issue(i + 2, p)

        @pl.when(i + 1 < _CPS)
        def _next_gather():
            e_wait(i + 1, 1 - p)
            fill_gi(1 - p)
            g_issue(1 - p)

        @plsc.parallel_loop(0, _B, 1, unroll=8)
        def _edge(e):
            g = e & ~15
            ewv = ewc[p, pl.ds(g, 16)]
            w = _lane_bcast(ewv, e - g)
            for j in range(4):
                a = w * wev[j] + bev[j]
                v = rows[p][e, pl.ds(16 * j, 16)]
                m = jnp.maximum(v + a, 1e-7)
                x = jnp.exp(m)
                och[p][e, pl.ds(16 * j, 16)] = x
                och[p][e, pl.ds(_HALF + 16 * j, 16)] = x * m
        s_issue(p)

    def pair(k, carry):
        step(2 * k, 0)
        step(2 * k + 1, 1)
        return carry
    lax.fori_loop(0, _CPS // 2, pair, 0)

    s_wait(0)
    s_wait(1)

    plsc.subcore_barrier()
    for k in range(_RPS // _ZR):
        r = row0 + k * _ZR
        pltpu.sync_copy(acc.at[pl.ds(r, _ZR)], out.at[c, pl.ds(r, _ZR)])

    @pl.when(s == 15)
    def _write_tail():
        pltpu.sync_copy(acc.at[pl.ds(N - 16, 16)],
                        out.at[c, pl.ds(N - 16, 16)])


@functools.cache
def _make_sc_agg():
    mesh = plsc.VectorSubcoreMesh(core_axis_name="c", subcore_axis_name="s",
                                  num_cores=2, num_subcores=16)
    return pl.kernel(
        _sc_agg_body,
        mesh=mesh,
        compiler_params=pltpu.CompilerParams(use_tc_tiling_on_sc=False),
        out_type=jax.ShapeDtypeStruct((2, N, HID), jnp.float32),
        scratch_types=[
            pltpu.VMEM((2, _B), jnp.int32),
            pltpu.VMEM((2, _B), jnp.int32),
            pltpu.VMEM((_B,), jnp.float32),
            pltpu.VMEM((_B,), jnp.float32),
            pltpu.VMEM((2, _B), jnp.float32),
            pltpu.VMEM((2, _B), jnp.int32),
            pltpu.VMEM((2, _B), jnp.int32),
            pltpu.VMEM((_B, _HALF), jnp.float32),
            pltpu.VMEM((_B, _HALF), jnp.float32),
            pltpu.VMEM((_B, HID), jnp.float32),
            pltpu.VMEM((_B, HID), jnp.float32),
            pltpu.VMEM((HID,), jnp.float32),
            pltpu.VMEM((_ZR, HID), jnp.float32),
            pltpu.VMEM_SHARED((N, HID), jnp.float32),
            pltpu.SemaphoreType.DMA,
            pltpu.SemaphoreType.DMA,
            pltpu.SemaphoreType.DMA,
            pltpu.SemaphoreType.DMA,
            pltpu.SemaphoreType.DMA,
            pltpu.SemaphoreType.DMA,
            pltpu.SemaphoreType.DMA,
            pltpu.SemaphoreType.DMA,
        ],
    )


def _pack_edges(src, dst, ew):
    edata = jnp.stack([src.reshape(-1, _B), dst.reshape(-1, _B)],
                      axis=1)
    return edata, ew.reshape(-1, _B)


def _aggregate(h_src, edata, ewr, we, be):
    bep = be + 1e-7
    wb = jnp.stack([
        jnp.concatenate([we[0, :_HALF], bep[:_HALF]]),
        jnp.concatenate([we[0, _HALF:], bep[_HALF:]]),
    ])
    return _make_sc_agg()(h_src.reshape(2 * N, _HALF), edata, ewr, wb)


def kernel(x_vals, x_cons, pe_vals, pe_cons, edge_index_v2c, edge_weight_v2c,
           edge_index_c2v, edge_weight_c2v, enc_vals_W, enc_vals_b,
           pe_vals_W1, pe_vals_b1, pe_vals_W2, pe_vals_b2,
           pred_vals_W1, pred_vals_b1, pred_vals_W2, pred_vals_b2,
           enc_cons_W, enc_cons_b,
           pe_cons_W1, pe_cons_b1, pe_cons_W2, pe_cons_b2,
           pred_cons_W1, pred_cons_b1, pred_cons_W2, pred_cons_b2,
           v2c0_We, v2c0_be, v2c0_W1, v2c0_b1, v2c0_W2, v2c0_b2,
           c2v0_We, c2v0_be, c2v0_W1, c2v0_b1, c2v0_W2, c2v0_b2,
           v2c1_We, v2c1_be, v2c1_W1, v2c1_b1, v2c1_W2, v2c1_b2,
           c2v1_We, c2v1_be, c2v1_W1, c2v1_b1, c2v1_W2, c2v1_b2):
    vals = _encoder(x_vals, pe_vals, enc_vals_W, enc_vals_b,
                    pe_vals_W1, pe_vals_b1, pe_vals_W2, pe_vals_b2)
    cons = _encoder(x_cons, pe_cons, enc_cons_W, enc_cons_b,
                    pe_cons_W1, pe_cons_b1, pe_cons_W2, pe_cons_b2)

    ed_v2c, ewr_v2c = _pack_edges(edge_index_v2c[0], edge_index_v2c[1],
                                  edge_weight_v2c[:, 0])
    ed_c2v, ewr_c2v = _pack_edges(edge_index_c2v[0], edge_index_c2v[1],
                                  edge_weight_c2v[:, 0])

    rounds = [
        ("v2c", v2c0_We, v2c0_be, v2c0_W1, v2c0_b1, v2c0_W2, v2c0_b2),
        ("c2v", c2v0_We, c2v0_be, c2v0_W1, c2v0_b1, c2v0_W2, c2v0_b2),
        ("v2c", v2c1_We, v2c1_be, v2c1_W1, v2c1_b1, v2c1_W2, v2c1_b2),
        ("c2v", c2v1_We, c2v1_be, c2v1_W1, c2v1_b1, c2v1_W2, c2v1_b2),
    ]
    hid_v, hid_c = [], []
    for d, we, be, w1, b1, w2, b2 in rounds:
        if d == "v2c":
            dn = _aggregate(vals, ed_v2c, ewr_v2c, we, be)
            cons = _round_update(dn, cons, w1, b1, w2, b2)
            hid_c.append(cons)
        else:
            dn = _aggregate(cons, ed_c2v, ewr_c2v, we, be)
            vals = _round_update(dn, vals, w1, b1, w2, b2)
            hid_v.append(vals)

    v2 = jnp.concatenate(hid_v, axis=0)
    c2 = jnp.concatenate(hid_c, axis=0)
    pv = _pred(v2, pred_vals_W1, pred_vals_b1, pred_vals_W2, pred_vals_b2)
    pc = _pred(c2, pred_cons_W1, pred_cons_b1, pred_cons_W2, pred_cons_b2)
    v = pv[:, 0].reshape(2, N).T
    c = pc[:, 0].reshape(2, N).T
    return (v, c)


# speedup vs baseline: 1.2492x; 1.0364x over previous
import functools

import jax
import jax.numpy as jnp
from jax import lax
from jax.experimental import pallas as pl
from jax.experimental.pallas import tpu as pltpu
from jax.experimental.pallas import tpu_sc as plsc

N = 10000
E = 320000
IN_SHAPE = 128
PE_DIM = 8
HID = 128

_BLK = 1000


def _enc_body(x_ref, pe_ref, w_ref, b_ref, w1_ref, b1_ref, w2_ref, b2_ref,
              out_ref):
    x = x_ref[...]
    pe = pe_ref[...]
    h1 = jnp.dot(x, w_ref[...], preferred_element_type=jnp.float32, precision=lax.Precision.HIGHEST) + b_ref[...]
    z = jnp.dot(pe, w1_ref[...], preferred_element_type=jnp.float32, precision=lax.Precision.HIGHEST)
    b1 = b1_ref[...]
    h = jax.nn.relu(z + b1) + jax.nn.relu(-z + b1)
    h2 = 0.5 * jnp.dot(h, w2_ref[...], preferred_element_type=jnp.float32, precision=lax.Precision.HIGHEST) \
        + b2_ref[...]
    out_ref[...] = jax.nn.relu(jnp.concatenate([h1, h2], axis=1))


def _encoder(x, pe, w, b, w1, b1, w2, b2):
    pe_p = jnp.pad(pe, ((0, 0), (0, IN_SHAPE - PE_DIM)))
    w1_p = jnp.pad(w1, ((0, IN_SHAPE - PE_DIM), (0, 0)))
    grid = (N // _BLK,)
    return pl.pallas_call(
        _enc_body,
        grid=grid,
        in_specs=[
            pl.BlockSpec((_BLK, IN_SHAPE), lambda i: (i, 0)),
            pl.BlockSpec((_BLK, IN_SHAPE), lambda i: (i, 0)),
            pl.BlockSpec((IN_SHAPE, HID // 2), lambda i: (0, 0)),
            pl.BlockSpec((1, HID // 2), lambda i: (0, 0)),
            pl.BlockSpec((IN_SHAPE, HID), lambda i: (0, 0)),
            pl.BlockSpec((1, HID), lambda i: (0, 0)),
            pl.BlockSpec((HID, HID // 2), lambda i: (0, 0)),
            pl.BlockSpec((1, HID // 2), lambda i: (0, 0)),
        ],
        out_specs=pl.BlockSpec((_BLK, HID), lambda i: (i, 0)),
        out_shape=jax.ShapeDtypeStruct((N, HID), jnp.float32),
    )(x, pe_p, w, b.reshape(1, -1), w1_p, b1.reshape(1, -1), w2,
      b2.reshape(1, -1))


def _round_body(dn_ref, xd_ref, w1_ref, b1_ref, w2_ref, b2_ref, out_ref):
    dn = dn_ref[...]
    den = jnp.concatenate([dn[0, :, :HID // 2], dn[1, :, :HID // 2]], axis=1)
    num = jnp.concatenate([dn[0, :, HID // 2:], dn[1, :, HID // 2:]], axis=1)
    agg = num / (den + 1e-16)
    o = agg + xd_ref[...]
    h = jax.nn.relu(
        jnp.dot(o, w1_ref[...], preferred_element_type=jnp.float32, precision=lax.Precision.HIGHEST)
        + b1_ref[...])
    out_ref[...] = jnp.dot(h, w2_ref[...],
                           preferred_element_type=jnp.float32, precision=lax.Precision.HIGHEST) + b2_ref[...]


def _round_update(dn, x_dst, w1, b1, w2, b2):
    grid = (N // _BLK,)
    return pl.pallas_call(
        _round_body,
        grid=grid,
        in_specs=[
            pl.BlockSpec((2, _BLK, HID), lambda i: (0, i, 0)),
            pl.BlockSpec((_BLK, HID), lambda i: (i, 0)),
            pl.BlockSpec((HID, 2 * HID), lambda i: (0, 0)),
            pl.BlockSpec((1, 2 * HID), lambda i: (0, 0)),
            pl.BlockSpec((2 * HID, HID), lambda i: (0, 0)),
            pl.BlockSpec((1, HID), lambda i: (0, 0)),
        ],
        out_specs=pl.BlockSpec((_BLK, HID), lambda i: (i, 0)),
        out_shape=jax.ShapeDtypeStruct((N, HID), jnp.float32),
    )(dn, x_dst, w1, b1.reshape(1, -1), w2, b2.reshape(1, -1))


def _pred_body(v_ref, w1_ref, b1_ref, w2_ref, b2_ref, out_ref):
    h = jax.nn.relu(
        jnp.dot(v_ref[...], w1_ref[...], preferred_element_type=jnp.float32)
        + b1_ref[...])
    out_ref[...] = jnp.dot(h, w2_ref[...],
                           preferred_element_type=jnp.float32) + b2_ref[...]


def _pred(v2, w1, b1, w2, b2):
    w2_p = jnp.pad(w2, ((0, 0), (0, 7)))
    b2_p = jnp.pad(b2, ((0, 7)))
    grid = (2 * N // _BLK,)
    return pl.pallas_call(
        _pred_body,
        grid=grid,
        in_specs=[
            pl.BlockSpec((_BLK, HID), lambda i: (i, 0)),
            pl.BlockSpec((HID, HID), lambda i: (0, 0)),
            pl.BlockSpec((1, HID), lambda i: (0, 0)),
            pl.BlockSpec((HID, 8), lambda i: (0, 0)),
            pl.BlockSpec((1, 8), lambda i: (0, 0)),
        ],
        out_specs=pl.BlockSpec((_BLK, 8), lambda i: (i, 0)),
        out_shape=jax.ShapeDtypeStruct((2 * N, 8), jnp.float32),
    )(v2, w1, b1.reshape(1, -1), w2_p, b2_p.reshape(1, -1))


_HALF = HID // 2
_B = 80
_NCHK = E // _B
_CPS = _NCHK // 16
_RPS = 624
_ZR = 48


def _lane_bcast(v, i):
    return lax.gather(
        v, jnp.full((16, 1), i, jnp.int32),
        lax.GatherDimensionNumbers(offset_dims=(), collapsed_slice_dims=(0,),
                                   start_index_map=(0,)),
        (1,), mode=lax.GatherScatterMode.PROMISE_IN_BOUNDS)


def _sc_agg_body(h2, edata, ewr, wbr, out,
                 eb0, eb1, ewb0, ewb1, ewc, dstv, gi, rows0, rows1,
                 och0, och1, wbv, zv, acc,
                 esem0, esem1, wsem0, wsem1, gsem0, gsem1, ssem0, ssem1):
    c = lax.axis_index("c")
    s = lax.axis_index("s")
    cbase = c * _HALF
    eb = (eb0, eb1)
    ewb = (ewb0, ewb1)
    rows = (rows0, rows1)
    och = (och0, och1)
    esem = (esem0, esem1)
    wsem = (wsem0, wsem1)
    gsem = (gsem0, gsem1)
    ssem = (ssem0, ssem1)

    def zrow(r, carry):
        for j in range(8):
            zv[r, pl.ds(16 * j, 16)] = jnp.zeros((16,), jnp.float32)
        return carry
    lax.fori_loop(0, _ZR, zrow, 0)
    row0 = s * _RPS
    for k in range(_RPS // _ZR):
        pltpu.sync_copy(zv, acc.at[pl.ds(row0 + k * _ZR, _ZR)])

    @pl.when(s == 15)
    def _zero_tail():
        pltpu.sync_copy(zv.at[pl.ds(0, 16)], acc.at[pl.ds(N - 16, 16)])
    plsc.subcore_barrier()

    pltpu.sync_copy(wbr.at[c], wbv)
    wev = [wbv[pl.ds(16 * j, 16)] for j in range(4)]
    bev = [wbv[pl.ds(_HALF + 16 * j, 16)] for j in range(4)]

    crow = s * _CPS

    def e_issue(i, p):
        pltpu.async_copy(edata.at[crow + i], eb[p], esem[p])
        pltpu.async_copy(ewr.at[crow + i], ewb[p], wsem[p])

    def e_wait(i, p):
        pltpu.make_async_copy(edata.at[crow + i], eb[p], esem[p]).wait()
        pltpu.make_async_copy(ewr.at[crow + i], ewb[p], wsem[p]).wait()

    def g_issue(p):
        pltpu.async_copy(h2.at[gi.at[p]], rows[p], gsem[p])

    def g_wait(p):
        pltpu.make_async_copy(h2.at[gi.at[p]], rows[p], gsem[p]).wait()

    def s_issue(p):
        pltpu.async_copy(och[p], acc.at[dstv.at[p]], ssem[p], add=True)

    def s_wait(p):
        pltpu.make_async_copy(och[p], acc.at[dstv.at[p]], ssem[p]).wait()

    _offs = [16 * t for t in range(_B // 16)] + (
        [] if _B % 16 == 0 else [_B - 16])

    def fill_gi(p):
        for o in _offs:
            sl = pl.ds(o, 16)
            gi[p, sl] = eb[p][0, sl] * 2 + c

    e_issue(0, 0)
    e_issue(1, 1)
    e_wait(0, 0)
    fill_gi(0)
    g_issue(0)

    def step(i, p):
        g_wait(p)

        @pl.when(i >= 2)
        def _drain():
            s_wait(p)

        for o in _offs:
            sl = pl.ds(o, 16)
            dstv[p, sl] = eb[p][1, sl]
            ewc[p, sl] = ewb[p][sl]

        @pl.when(i + 2 < _CPS)
        def _stage():
            e_issue(i + 2, p)

        @pl.when(i + 1 < _CPS)
        def _next_gather():
            e_wait(i + 1, 1 - p)
            fill_gi(1 - p)
            g_issue(1 - p)

        @plsc.parallel_loop(0, _B, 1, unroll=8)
        def _edge(e):
            g = e & ~15
            ewv = ewc[p, pl.ds(g, 16)]
            w = _lane_bcast(ewv, e - g)
            for j in range(4):
                a = w * wev[j] + bev[j]
                v = rows[p][e, pl.ds(16 * j, 16)]
                m = jnp.maximum(v + a, 1e-7)
                x = jnp.exp(m)
                och[p][e, pl.ds(16 * j, 16)] = x
                och[p][e, pl.ds(_HALF + 16 * j, 16)] = x * m
        s_issue(p)

    def pair(k, carry):
        step(2 * k, 0)
        step(2 * k + 1, 1)
        return carry
    lax.fori_loop(0, _CPS // 2, pair, 0)

    s_wait(0)
    s_wait(1)

    plsc.subcore_barrier()
    for k in range(_RPS // _ZR):
        r = row0 + k * _ZR
        pltpu.sync_copy(acc.at[pl.ds(r, _ZR)], out.at[c, pl.ds(r, _ZR)])

    @pl.when(s == 15)
    def _write_tail():
        pltpu.sync_copy(acc.at[pl.ds(N - 16, 16)],
                        out.at[c, pl.ds(N - 16, 16)])


@functools.cache
def _make_sc_agg():
    mesh = plsc.VectorSubcoreMesh(core_axis_name="c", subcore_axis_name="s",
                                  num_cores=2, num_subcores=16)
    return pl.kernel(
        _sc_agg_body,
        mesh=mesh,
        compiler_params=pltpu.CompilerParams(use_tc_tiling_on_sc=False),
        out_type=jax.ShapeDtypeStruct((2, N, HID), jnp.float32),
        scratch_types=[
            pltpu.VMEM((2, _B), jnp.int32),
            pltpu.VMEM((2, _B), jnp.int32),
            pltpu.VMEM((_B,), jnp.float32),
            pltpu.VMEM((_B,), jnp.float32),
            pltpu.VMEM((2, _B), jnp.float32),
            pltpu.VMEM((2, _B), jnp.int32),
            pltpu.VMEM((2, _B), jnp.int32),
            pltpu.VMEM((_B, _HALF), jnp.float32),
            pltpu.VMEM((_B, _HALF), jnp.float32),
            pltpu.VMEM((_B, HID), jnp.float32),
            pltpu.VMEM((_B, HID), jnp.float32),
            pltpu.VMEM((HID,), jnp.float32),
            pltpu.VMEM((_ZR, HID), jnp.float32),
            pltpu.VMEM_SHARED((N, HID), jnp.float32),
            pltpu.SemaphoreType.DMA,
            pltpu.SemaphoreType.DMA,
            pltpu.SemaphoreType.DMA,
            pltpu.SemaphoreType.DMA,
            pltpu.SemaphoreType.DMA,
            pltpu.SemaphoreType.DMA,
            pltpu.SemaphoreType.DMA,
            pltpu.SemaphoreType.DMA,
        ],
    )


def _pack_edges(src, dst, ew):
    edata = jnp.stack([src.reshape(-1, _B), dst.reshape(-1, _B)],
                      axis=1)
    return edata, ew.reshape(-1, _B)


def _aggregate(h_src, edata, ewr, we, be):
    bep = be + 1e-7
    wb = jnp.stack([
        jnp.concatenate([we[0, :_HALF], bep[:_HALF]]),
        jnp.concatenate([we[0, _HALF:], bep[_HALF:]]),
    ])
    return _make_sc_agg()(h_src.reshape(2 * N, _HALF), edata, ewr, wb)


def kernel(x_vals, x_cons, pe_vals, pe_cons, edge_index_v2c, edge_weight_v2c,
           edge_index_c2v, edge_weight_c2v, enc_vals_W, enc_vals_b,
           pe_vals_W1, pe_vals_b1, pe_vals_W2, pe_vals_b2,
           pred_vals_W1, pred_vals_b1, pred_vals_W2, pred_vals_b2,
           enc_cons_W, enc_cons_b,
           pe_cons_W1, pe_cons_b1, pe_cons_W2, pe_cons_b2,
           pred_cons_W1, pred_cons_b1, pred_cons_W2, pred_cons_b2,
           v2c0_We, v2c0_be, v2c0_W1, v2c0_b1, v2c0_W2, v2c0_b2,
           c2v0_We, c2v0_be, c2v0_W1, c2v0_b1, c2v0_W2, c2v0_b2,
           v2c1_We, v2c1_be, v2c1_W1, v2c1_b1, v2c1_W2, v2c1_b2,
           c2v1_We, c2v1_be, c2v1_W1, c2v1_b1, c2v1_W2, c2v1_b2):
    vals = _encoder(x_vals, pe_vals, enc_vals_W, enc_vals_b,
                    pe_vals_W1, pe_vals_b1, pe_vals_W2, pe_vals_b2)
    cons = _encoder(x_cons, pe_cons, enc_cons_W, enc_cons_b,
                    pe_cons_W1, pe_cons_b1, pe_cons_W2, pe_cons_b2)

    ed_v2c, ewr_v2c = _pack_edges(edge_index_v2c[0], edge_index_v2c[1],
                                  edge_weight_v2c[:, 0])
    ed_c2v, ewr_c2v = _pack_edges(edge_index_c2v[0], edge_index_c2v[1],
                                  edge_weight_c2v[:, 0])

    rounds = [
        ("v2c", v2c0_We, v2c0_be, v2c0_W1, v2c0_b1, v2c0_W2, v2c0_b2),
        ("c2v", c2v0_We, c2v0_be, c2v0_W1, c2v0_b1, c2v0_W2, c2v0_b2),
        ("v2c", v2c1_We, v2c1_be, v2c1_W1, v2c1_b1, v2c1_W2, v2c1_b2),
        ("c2v", c2v1_We, c2v1_be, c2v1_W1, c2v1_b1, c2v1_W2, c2v1_b2),
    ]
    hid_v, hid_c = [], []
    for d, we, be, w1, b1, w2, b2 in rounds:
        if d == "v2c":
            dn = _aggregate(vals, ed_v2c, ewr_v2c, we, be)
            cons = _round_update(dn, cons, w1, b1, w2, b2)
            hid_c.append(cons)
        else:
            dn = _aggregate(cons, ed_c2v, ewr_c2v, we, be)
            vals = _round_update(dn, vals, w1, b1, w2, b2)
            hid_v.append(vals)

    v2 = jnp.concatenate(hid_v, axis=0)
    c2 = jnp.concatenate(hid_c, axis=0)
    pv = _pred(v2, pred_vals_W1, pred_vals_b1, pred_vals_W2, pred_vals_b2)
    pc = _pred(c2, pred_cons_W1, pred_cons_b1, pred_cons_W2, pred_cons_b2)
    v = pv[:, 0].reshape(2, N).T
    c = pc[:, 0].reshape(2, N).T
    return (v, c)


# speedup vs baseline: 1.3267x; 1.0620x over previous
import functools

import jax
import jax.numpy as jnp
from jax import lax
from jax.experimental import pallas as pl
from jax.experimental.pallas import tpu as pltpu
from jax.experimental.pallas import tpu_sc as plsc

N = 10000
E = 320000
IN_SHAPE = 128
PE_DIM = 8
HID = 128

_BLK = 1000


def _enc_body(x_ref, pe_ref, w_ref, b_ref, w1_ref, b1_ref, w2_ref, b2_ref,
              out_ref):
    x = x_ref[...]
    pe = pe_ref[...]
    h1 = jnp.dot(x, w_ref[...], preferred_element_type=jnp.float32, precision=lax.Precision.HIGHEST) + b_ref[...]
    z = jnp.dot(pe, w1_ref[...], preferred_element_type=jnp.float32, precision=lax.Precision.HIGHEST)
    b1 = b1_ref[...]
    h = jax.nn.relu(z + b1) + jax.nn.relu(-z + b1)
    h2 = 0.5 * jnp.dot(h, w2_ref[...], preferred_element_type=jnp.float32, precision=lax.Precision.HIGHEST) \
        + b2_ref[...]
    out_ref[...] = jax.nn.relu(jnp.concatenate([h1, h2], axis=1))


def _encoder(x, pe, w, b, w1, b1, w2, b2):
    pe_p = jnp.pad(pe, ((0, 0), (0, IN_SHAPE - PE_DIM)))
    w1_p = jnp.pad(w1, ((0, IN_SHAPE - PE_DIM), (0, 0)))
    grid = (N // _BLK,)
    return pl.pallas_call(
        _enc_body,
        grid=grid,
        in_specs=[
            pl.BlockSpec((_BLK, IN_SHAPE), lambda i: (i, 0)),
            pl.BlockSpec((_BLK, IN_SHAPE), lambda i: (i, 0)),
            pl.BlockSpec((IN_SHAPE, HID // 2), lambda i: (0, 0)),
            pl.BlockSpec((1, HID // 2), lambda i: (0, 0)),
            pl.BlockSpec((IN_SHAPE, HID), lambda i: (0, 0)),
            pl.BlockSpec((1, HID), lambda i: (0, 0)),
            pl.BlockSpec((HID, HID // 2), lambda i: (0, 0)),
            pl.BlockSpec((1, HID // 2), lambda i: (0, 0)),
        ],
        out_specs=pl.BlockSpec((_BLK, HID), lambda i: (i, 0)),
        out_shape=jax.ShapeDtypeStruct((N, HID), jnp.float32),
    )(x, pe_p, w, b.reshape(1, -1), w1_p, b1.reshape(1, -1), w2,
      b2.reshape(1, -1))


def _round_body(dn_ref, xd_ref, w1_ref, b1_ref, w2_ref, b2_ref, out_ref):
    dn = dn_ref[...]
    den = jnp.concatenate([dn[0, :, :HID // 2], dn[1, :, :HID // 2]], axis=1)
    num = jnp.concatenate([dn[0, :, HID // 2:], dn[1, :, HID // 2:]], axis=1)
    agg = num / (den + 1e-16)
    o = agg + xd_ref[...]
    h = jax.nn.relu(
        jnp.dot(o, w1_ref[...], preferred_element_type=jnp.float32, precision=lax.Precision.HIGHEST)
        + b1_ref[...])
    out_ref[...] = jnp.dot(h, w2_ref[...],
                           preferred_element_type=jnp.float32) + b2_ref[...]


def _round_update(dn, x_dst, w1, b1, w2, b2):
    grid = (N // _BLK,)
    return pl.pallas_call(
        _round_body,
        grid=grid,
        in_specs=[
            pl.BlockSpec((2, _BLK, HID), lambda i: (0, i, 0)),
            pl.BlockSpec((_BLK, HID), lambda i: (i, 0)),
            pl.BlockSpec((HID, 2 * HID), lambda i: (0, 0)),
            pl.BlockSpec((1, 2 * HID), lambda i: (0, 0)),
            pl.BlockSpec((2 * HID, HID), lambda i: (0, 0)),
            pl.BlockSpec((1, HID), lambda i: (0, 0)),
        ],
        out_specs=pl.BlockSpec((_BLK, HID), lambda i: (i, 0)),
        out_shape=jax.ShapeDtypeStruct((N, HID), jnp.float32),
    )(dn, x_dst, w1, b1.reshape(1, -1), w2, b2.reshape(1, -1))


def _pred_body(v_ref, w1_ref, b1_ref, w2_ref, b2_ref, out_ref):
    h = jax.nn.relu(
        jnp.dot(v_ref[...], w1_ref[...], preferred_element_type=jnp.float32)
        + b1_ref[...])
    out_ref[...] = jnp.dot(h, w2_ref[...],
                           preferred_element_type=jnp.float32) + b2_ref[...]


def _pred(v2, w1, b1, w2, b2):
    w2_p = jnp.pad(w2, ((0, 0), (0, 7)))
    b2_p = jnp.pad(b2, ((0, 7)))
    grid = (2 * N // _BLK,)
    return pl.pallas_call(
        _pred_body,
        grid=grid,
        in_specs=[
            pl.BlockSpec((_BLK, HID), lambda i: (i, 0)),
            pl.BlockSpec((HID, HID), lambda i: (0, 0)),
            pl.BlockSpec((1, HID), lambda i: (0, 0)),
            pl.BlockSpec((HID, 8), lambda i: (0, 0)),
            pl.BlockSpec((1, 8), lambda i: (0, 0)),
        ],
        out_specs=pl.BlockSpec((_BLK, 8), lambda i: (i, 0)),
        out_shape=jax.ShapeDtypeStruct((2 * N, 8), jnp.float32),
    )(v2, w1, b1.reshape(1, -1), w2_p, b2_p.reshape(1, -1))


_HALF = HID // 2
_B = 80
_NCHK = E // _B
_CPS = _NCHK // 16
_RPS = 624
_ZR = 48


def _lane_bcast(v, i):
    return lax.gather(
        v, jnp.full((16, 1), i, jnp.int32),
        lax.GatherDimensionNumbers(offset_dims=(), collapsed_slice_dims=(0,),
                                   start_index_map=(0,)),
        (1,), mode=lax.GatherScatterMode.PROMISE_IN_BOUNDS)


def _sc_agg_body(h2, edata, ewr, wbr, out,
                 eb0, eb1, ewb0, ewb1, ewc, dstv, gi, rows0, rows1,
                 och0, och1, wbv, zv, acc,
                 esem0, esem1, wsem0, wsem1, gsem0, gsem1, ssem0, ssem1):
    c = lax.axis_index("c")
    s = lax.axis_index("s")
    cbase = c * _HALF
    eb = (eb0, eb1)
    ewb = (ewb0, ewb1)
    rows = (rows0, rows1)
    och = (och0, och1)
    esem = (esem0, esem1)
    wsem = (wsem0, wsem1)
    gsem = (gsem0, gsem1)
    ssem = (ssem0, ssem1)

    def zrow(r, carry):
        for j in range(8):
            zv[r, pl.ds(16 * j, 16)] = jnp.zeros((16,), jnp.float32)
        return carry
    lax.fori_loop(0, _ZR, zrow, 0)
    row0 = s * _RPS
    for k in range(_RPS // _ZR):
        pltpu.sync_copy(zv, acc.at[pl.ds(row0 + k * _ZR, _ZR)])

    @pl.when(s == 15)
    def _zero_tail():
        pltpu.sync_copy(zv.at[pl.ds(0, 16)], acc.at[pl.ds(N - 16, 16)])
    plsc.subcore_barrier()

    pltpu.sync_copy(wbr.at[c], wbv)
    wev = [wbv[pl.ds(16 * j, 16)] for j in range(4)]
    bev = [wbv[pl.ds(_HALF + 16 * j, 16)] for j in range(4)]

    crow = s * _CPS

    def e_issue(i, p):
        pltpu.async_copy(edata.at[crow + i], eb[p], esem[p])
        pltpu.async_copy(ewr.at[crow + i], ewb[p], wsem[p])

    def e_wait(i, p):
        pltpu.make_async_copy(edata.at[crow + i], eb[p], esem[p]).wait()
        pltpu.make_async_copy(ewr.at[crow + i], ewb[p], wsem[p]).wait()

    def g_issue(p):
        pltpu.async_copy(h2.at[gi.at[p]], rows[p], gsem[p])

    def g_wait(p):
        pltpu.make_async_copy(h2.at[gi.at[p]], rows[p], gsem[p]).wait()

    def s_issue(p):
        pltpu.async_copy(och[p], acc.at[dstv.at[p]], ssem[p], add=True)

    def s_wait(p):
        pltpu.make_async_copy(och[p], acc.at[dstv.at[p]], ssem[p]).wait()

    _offs = [16 * t for t in range(_B // 16)] + (
        [] if _B % 16 == 0 else [_B - 16])

    def fill_gi(p):
        for o in _offs:
            sl = pl.ds(o, 16)
            gi[p, sl] = eb[p][0, sl] * 2 + c

    e_issue(0, 0)
    e_issue(1, 1)
    e_wait(0, 0)
    fill_gi(0)
    g_issue(0)

    def step(i, p):
        g_wait(p)

        @pl.when(i >= 2)
        def _drain():
            s_wait(p)

        for o in _offs:
            sl = pl.ds(o, 16)
            dstv[p, sl] = eb[p][1, sl]
            ewc[p, sl] = ewb[p][sl]

        @pl.when(i + 2 < _CPS)
        def _stage():
            e_issue(i + 2, p)

        @pl.when(i + 1 < _CPS)
        def _next_gather():
            e_wait(i + 1, 1 - p)
            fill_gi(1 - p)
            g_issue(1 - p)

        @plsc.parallel_loop(0, _B, 1, unroll=8)
        def _edge(e):
            g = e & ~15
            ewv = ewc[p, pl.ds(g, 16)]
            w = _lane_bcast(ewv, e - g)
            for j in range(4):
                a = w * wev[j] + bev[j]
                v = rows[p][e, pl.ds(16 * j, 16)]
                m = jnp.maximum(v + a, 1e-7)
                x = jnp.exp(m)
                och[p][e, pl.ds(16 * j, 16)] = x
                och[p][e, pl.ds(_HALF + 16 * j, 16)] = x * m
        s_issue(p)

    def pair(k, carry):
        step(2 * k, 0)
        step(2 * k + 1, 1)
        return carry
    lax.fori_loop(0, _CPS // 2, pair, 0)

    s_wait(0)
    s_wait(1)

    plsc.subcore_barrier()
    for k in range(_RPS // _ZR):
        r = row0 + k * _ZR
        pltpu.sync_copy(acc.at[pl.ds(r, _ZR)], out.at[c, pl.ds(r, _ZR)])

    @pl.when(s == 15)
    def _write_tail():
        pltpu.sync_copy(acc.at[pl.ds(N - 16, 16)],
                        out.at[c, pl.ds(N - 16, 16)])


@functools.cache
def _make_sc_agg():
    mesh = plsc.VectorSubcoreMesh(core_axis_name="c", subcore_axis_name="s",
                                  num_cores=2, num_subcores=16)
    return pl.kernel(
        _sc_agg_body,
        mesh=mesh,
        compiler_params=pltpu.CompilerParams(use_tc_tiling_on_sc=False),
        out_type=jax.ShapeDtypeStruct((2, N, HID), jnp.float32),
        scratch_types=[
            pltpu.VMEM((2, _B), jnp.int32),
            pltpu.VMEM((2, _B), jnp.int32),
            pltpu.VMEM((_B,), jnp.float32),
            pltpu.VMEM((_B,), jnp.float32),
            pltpu.VMEM((2, _B), jnp.float32),
            pltpu.VMEM((2, _B), jnp.int32),
            pltpu.VMEM((2, _B), jnp.int32),
            pltpu.VMEM((_B, _HALF), jnp.float32),
            pltpu.VMEM((_B, _HALF), jnp.float32),
            pltpu.VMEM((_B, HID), jnp.float32),
            pltpu.VMEM((_B, HID), jnp.float32),
            pltpu.VMEM((HID,), jnp.float32),
            pltpu.VMEM((_ZR, HID), jnp.float32),
            pltpu.VMEM_SHARED((N, HID), jnp.float32),
            pltpu.SemaphoreType.DMA,
            pltpu.SemaphoreType.DMA,
            pltpu.SemaphoreType.DMA,
            pltpu.SemaphoreType.DMA,
            pltpu.SemaphoreType.DMA,
            pltpu.SemaphoreType.DMA,
            pltpu.SemaphoreType.DMA,
            pltpu.SemaphoreType.DMA,
        ],
    )


def _pack_edges(src, dst, ew):
    edata = jnp.stack([src.reshape(-1, _B), dst.reshape(-1, _B)],
                      axis=1)
    return edata, ew.reshape(-1, _B)


def _aggregate(h_src, edata, ewr, we, be):
    bep = be + 1e-7
    wb = jnp.stack([
        jnp.concatenate([we[0, :_HALF], bep[:_HALF]]),
        jnp.concatenate([we[0, _HALF:], bep[_HALF:]]),
    ])
    return _make_sc_agg()(h_src.reshape(2 * N, _HALF), edata, ewr, wb)


def kernel(x_vals, x_cons, pe_vals, pe_cons, edge_index_v2c, edge_weight_v2c,
           edge_index_c2v, edge_weight_c2v, enc_vals_W, enc_vals_b,
           pe_vals_W1, pe_vals_b1, pe_vals_W2, pe_vals_b2,
           pred_vals_W1, pred_vals_b1, pred_vals_W2, pred_vals_b2,
           enc_cons_W, enc_cons_b,
           pe_cons_W1, pe_cons_b1, pe_cons_W2, pe_cons_b2,
           pred_cons_W1, pred_cons_b1, pred_cons_W2, pred_cons_b2,
           v2c0_We, v2c0_be, v2c0_W1, v2c0_b1, v2c0_W2, v2c0_b2,
           c2v0_We, c2v0_be, c2v0_W1, c2v0_b1, c2v0_W2, c2v0_b2,
           v2c1_We, v2c1_be, v2c1_W1, v2c1_b1, v2c1_W2, v2c1_b2,
           c2v1_We, c2v1_be, c2v1_W1, c2v1_b1, c2v1_W2, c2v1_b2):
    vals = _encoder(x_vals, pe_vals, enc_vals_W, enc_vals_b,
                    pe_vals_W1, pe_vals_b1, pe_vals_W2, pe_vals_b2)
    cons = _encoder(x_cons, pe_cons, enc_cons_W, enc_cons_b,
                    pe_cons_W1, pe_cons_b1, pe_cons_W2, pe_cons_b2)

    ed_v2c, ewr_v2c = _pack_edges(edge_index_v2c[0], edge_index_v2c[1],
                                  edge_weight_v2c[:, 0])
    ed_c2v, ewr_c2v = _pack_edges(edge_index_c2v[0], edge_index_c2v[1],
                                  edge_weight_c2v[:, 0])

    rounds = [
        ("v2c", v2c0_We, v2c0_be, v2c0_W1, v2c0_b1, v2c0_W2, v2c0_b2),
        ("c2v", c2v0_We, c2v0_be, c2v0_W1, c2v0_b1, c2v0_W2, c2v0_b2),
        ("v2c", v2c1_We, v2c1_be, v2c1_W1, v2c1_b1, v2c1_W2, v2c1_b2),
        ("c2v", c2v1_We, c2v1_be, c2v1_W1, c2v1_b1, c2v1_W2, c2v1_b2),
    ]
    hid_v, hid_c = [], []
    for d, we, be, w1, b1, w2, b2 in rounds:
        if d == "v2c":
            dn = _aggregate(vals, ed_v2c, ewr_v2c, we, be)
            cons = _round_update(dn, cons, w1, b1, w2, b2)
            hid_c.append(cons)
        else:
            dn = _aggregate(cons, ed_c2v, ewr_c2v, we, be)
            vals = _round_update(dn, vals, w1, b1, w2, b2)
            hid_v.append(vals)

    v2 = jnp.concatenate(hid_v, axis=0)
    c2 = jnp.concatenate(hid_c, axis=0)
    pv = _pred(v2, pred_vals_W1, pred_vals_b1, pred_vals_W2, pred_vals_b2)
    pc = _pred(c2, pred_cons_W1, pred_cons_b1, pred_cons_W2, pred_cons_b2)
    v = pv[:, 0].reshape(2, N).T
    c = pc[:, 0].reshape(2, N).T
    return (v, c)
